# baseline (speedup 1.0000x reference)
import functools

import jax
import jax.numpy as jnp
from jax.experimental import pallas as pl
from jax.experimental.pallas import tpu as pltpu

E = 16
D = 1024
H = 4096
R = 4
BATCH = 32

HB = 512


def _gate_kernel(x_ref, wg_ref, bg_ref, w_ref, idx_ref):
    x = x_ref[...]
    logits = jax.lax.dot_general(
        x, wg_ref[...], (((1,), (1,)), ((), ())),
        preferred_element_type=jnp.float32) + bg_ref[...]
    m = jnp.max(logits, axis=-1, keepdims=True)
    ex = jnp.exp(logits - m)
    probs = ex / jnp.sum(ex, axis=-1, keepdims=True)
    w_ref[...] = jnp.max(probs, axis=-1, keepdims=True)
    idx_ref[...] = jnp.argmax(probs, axis=-1, keepdims=True).astype(jnp.int32)


def _ffn_kernel(idx_ref, x_ref, w1_ref, b1_ref, a1_ref, bb1_ref,
                w2_ref, b2_ref, a2_ref, bb2_ref, w_ref, out_ref, acc_ref):
    i = pl.program_id(0)
    nsteps = pl.num_programs(0)
    x = x_ref[...]
    t1 = jax.lax.dot_general(x, bb1_ref[0], (((1,), (1,)), ((), ())),
                             preferred_element_type=jnp.float32)
    h = jax.lax.dot_general(x, w1_ref[0], (((1,), (1,)), ((), ())),
                            preferred_element_type=jnp.float32)
    h = h + jax.lax.dot_general(t1, a1_ref[0], (((1,), (1,)), ((), ())),
                                preferred_element_type=jnp.float32)
    h = jnp.maximum(h + b1_ref[0], 0.0)
    p = jax.lax.dot_general(h, w2_ref[0], (((1,), (1,)), ((), ())),
                            preferred_element_type=jnp.float32)
    t2 = jax.lax.dot_general(h, bb2_ref[0], (((1,), (1,)), ((), ())),
                             preferred_element_type=jnp.float32)
    p = p + jax.lax.dot_general(t2, a2_ref[0], (((1,), (1,)), ((), ())),
                                preferred_element_type=jnp.float32)

    @pl.when(i == 0)
    def _():
        acc_ref[...] = p

    @pl.when(i > 0)
    def _():
        acc_ref[...] = acc_ref[...] + p

    @pl.when(i == nsteps - 1)
    def _():
        out_ref[...] = (acc_ref[...] + b2_ref[0]) * w_ref[...]


@jax.jit
def kernel(x, Wg, bg, W1, b1, A1, B1, W2, b2, A2, B2):
    topw, topi = pl.pallas_call(
        _gate_kernel,
        out_shape=(
            jax.ShapeDtypeStruct((BATCH, 1), jnp.float32),
            jax.ShapeDtypeStruct((BATCH, 1), jnp.int32),
        ),
    )(x, Wg, bg.reshape(1, E))

    e_idx = topi[0]

    grid_spec = pltpu.PrefetchScalarGridSpec(
        num_scalar_prefetch=1,
        grid=(H // HB,),
        in_specs=[
            pl.BlockSpec((BATCH, D), lambda i, e: (0, 0)),
            pl.BlockSpec((1, HB, D), lambda i, e: (e[0], i, 0)),
            pl.BlockSpec((1, 1, HB), lambda i, e: (e[0], 0, i)),
            pl.BlockSpec((1, HB, R), lambda i, e: (e[0], i, 0)),
            pl.BlockSpec((1, R, D), lambda i, e: (e[0], 0, 0)),
            pl.BlockSpec((1, D, HB), lambda i, e: (e[0], 0, i)),
            pl.BlockSpec((1, 1, D), lambda i, e: (e[0], 0, 0)),
            pl.BlockSpec((1, D, R), lambda i, e: (e[0], 0, 0)),
            pl.BlockSpec((1, R, HB), lambda i, e: (e[0], 0, i)),
            pl.BlockSpec((BATCH, 1), lambda i, e: (0, 0)),
        ],
        out_specs=pl.BlockSpec((BATCH, D), lambda i, e: (0, 0)),
        scratch_shapes=[pltpu.VMEM((BATCH, D), jnp.float32)],
    )
    out = pl.pallas_call(
        _ffn_kernel,
        grid_spec=grid_spec,
        out_shape=jax.ShapeDtypeStruct((BATCH, D), jnp.float32),
    )(e_idx, x, W1, b1.reshape(E, 1, H), A1, B1, W2,
      b2.reshape(E, 1, D), A2, B2, topw)
    return (out, topi)
